# Initial kernel scaffold; baseline (speedup 1.0000x reference)
import jax
import jax.numpy as jnp
from jax.experimental import pallas as pl


def kernel(z, codebooks):
    raise NotImplementedError("write your pallas kernel here")


# speedup vs baseline: 1.2601x; 1.2601x over previous
import functools

import jax
import jax.numpy as jnp
from jax import lax
from jax.experimental import pallas as pl
from jax.experimental.pallas import tpu as pltpu
from jax.experimental.pallas import tpu_sc as plsc

NL = 8
K = 8192
D = 256
T = 4096
TM = 1024
KB = 2048
NKB = K // KB

NW = 32
BPW = T // NW


def argmin_body(x2_ref, y2_ref, r_ref, w_ref, idx_ref, rmin_ref, rarg_ref):
    k = pl.program_id(1)

    @pl.when(k == 0)
    def _init():
        rmin_ref[...] = jnp.full((TM, 1), jnp.inf, dtype=jnp.float32)
        rarg_ref[...] = jnp.zeros((TM, 1), dtype=jnp.int32)

    s = lax.dot_general(r_ref[...], w_ref[...], (((1,), (1,)), ((), ())),
                        preferred_element_type=jnp.float32)
    d = (x2_ref[...] + y2_ref[...]) - 2.0 * s
    m = jnp.min(d, axis=1, keepdims=True)
    iota = lax.broadcasted_iota(jnp.int32, (TM, KB), 1)
    loc = jnp.min(jnp.where(d == m, iota, KB), axis=1, keepdims=True)
    cand = loc + k * KB
    better = m < rmin_ref[...]
    rarg_ref[...] = jnp.where(better, cand, rarg_ref[...])
    rmin_ref[...] = jnp.where(better, m, rmin_ref[...])

    @pl.when(k == NKB - 1)
    def _flush():
        idx_ref[...] = rarg_ref[...][:, 0]


dist_argmin = pl.pallas_call(
    argmin_body,
    grid=(T // TM, NKB),
    in_specs=[
        pl.BlockSpec((TM, 1), lambda t, k: (t, 0)),
        pl.BlockSpec((1, KB), lambda t, k: (0, k)),
        pl.BlockSpec((TM, D), lambda t, k: (t, 0)),
        pl.BlockSpec((KB, D), lambda t, k: (k, 0)),
    ],
    out_specs=pl.BlockSpec((TM,), lambda t, k: (t,)),
    out_shape=jax.ShapeDtypeStruct((T,), jnp.int32),
    scratch_shapes=[
        pltpu.VMEM((TM, 1), jnp.float32),
        pltpu.VMEM((TM, 1), jnp.int32),
    ],
    compiler_params=pltpu.CompilerParams(
        dimension_semantics=("parallel", "arbitrary"),
    ),
)


@functools.lru_cache(maxsize=1)
def _sc_gather():
    @functools.partial(
        pl.kernel,
        mesh=plsc.VectorSubcoreMesh(core_axis_name="c", subcore_axis_name="s"),
        out_type=jax.ShapeDtypeStruct((T, D), jnp.float32),
        scratch_types=[
            pltpu.VMEM((BPW,), jnp.int32),
            pltpu.VMEM((BPW, D), jnp.float32),
            pltpu.SemaphoreType.DMA,
        ],
    )
    def sc_gather(cb_hbm, gidx_hbm, out_hbm, idx_v, rows_v, sem):
        wid = lax.axis_index("s") * 2 + lax.axis_index("c")
        base = wid * BPW
        pltpu.sync_copy(gidx_hbm.at[pl.ds(base, BPW)], idx_v)
        pltpu.async_copy(cb_hbm.at[idx_v], rows_v, sem).wait()
        pltpu.sync_copy(rows_v, out_hbm.at[pl.ds(base, BPW)])

    return sc_gather


def kernel(z, codebooks):
    batch, dim, time = z.shape
    zt = jnp.transpose(z, (0, 2, 1))
    residual = zt.reshape(T, D)
    cb_flat = codebooks.reshape(NL * K, D)

    z_q = jnp.zeros_like(residual)
    loss = jnp.zeros((), dtype=jnp.float32)
    codes = []
    for layer in range(NL):
        W = codebooks[layer]
        x2 = jnp.sum(residual ** 2, axis=1, keepdims=True)
        y2 = jnp.sum(W ** 2, axis=1)[None, :]
        idx = dist_argmin(x2, y2, residual, W)
        q = _sc_gather()(cb_flat, idx + layer * K)
        loss = loss + jnp.mean((residual - q) ** 2)
        q_ste = residual + (q - residual)
        z_q = z_q + q_ste
        residual = residual - q_ste
        codes.append(idx.reshape(batch, time))

    z_q_out = jnp.transpose(z_q.reshape(batch, time, dim), (0, 2, 1))
    all_codes = jnp.stack(codes, axis=0)
    return (z_q_out, all_codes, loss, loss, loss + loss)


# speedup vs baseline: 1.2703x; 1.0081x over previous
import functools

import jax
import jax.numpy as jnp
from jax import lax
from jax.experimental import pallas as pl
from jax.experimental.pallas import tpu as pltpu
from jax.experimental.pallas import tpu_sc as plsc

NL = 8
K = 8192
D = 256
T = 4096
TM = 1024
KB = 2048
NKB = K // KB

NW = 32
BPW = T // NW


def _argmin_block(k, r, w_ref, x2h_ref, y2h_ref, iota_ref, rmin_ref, rarg_ref,
                  idx_ref):
    w = w_ref[...]

    @pl.when(pl.program_id(0) == 0)
    def _y2():
        y2h_ref[:, pl.ds(k * KB, KB)] = (jnp.sum(w * w, axis=1) * 0.5)[None, :]

    @pl.when((pl.program_id(0) == 0) & (k == 0))
    def _iota():
        iota_ref[...] = lax.broadcasted_iota(
            jnp.int32, (1, KB), 1).astype(jnp.float32)

    s = lax.dot_general(r, w, (((1,), (1,)), ((), ())),
                        preferred_element_type=jnp.float32)
    d = (x2h_ref[...] + y2h_ref[:, pl.ds(k * KB, KB)]) - s
    m = jnp.min(d, axis=1, keepdims=True)
    loc = jnp.min(jnp.where(d == m, iota_ref[...], float(KB)), axis=1,
                  keepdims=True)
    cand = loc.astype(jnp.int32) + k * KB
    better = m < rmin_ref[...]
    rarg_ref[...] = jnp.where(better, cand, rarg_ref[...])
    rmin_ref[...] = jnp.where(better, m, rmin_ref[...])

    @pl.when(k == NKB - 1)
    def _flush():
        idx_ref[...] = rarg_ref[...][:, 0]


def _init_minmax(rmin_ref, rarg_ref):
    rmin_ref[...] = jnp.full((TM, 1), jnp.inf, dtype=jnp.float32)
    rarg_ref[...] = jnp.zeros((TM, 1), dtype=jnp.int32)


def layer0_body(r_ref, w_ref, idx_ref, x2h_ref, y2h_ref, iota_ref, rmin_ref,
                rarg_ref):
    k = pl.program_id(1)

    @pl.when(k == 0)
    def _init():
        r = r_ref[...]
        x2h_ref[...] = jnp.sum(r * r, axis=1, keepdims=True) * 0.5
        _init_minmax(rmin_ref, rarg_ref)

    _argmin_block(k, r_ref[...], w_ref, x2h_ref, y2h_ref, iota_ref,
                  rmin_ref, rarg_ref, idx_ref)


def fused_body(rprev_ref, qprev_ref, zqprev_ref, w_ref,
               idx_ref, rnew_ref, zqnew_ref,
               x2h_ref, y2h_ref, iota_ref, rmin_ref, rarg_ref):
    k = pl.program_id(1)

    @pl.when(k == 0)
    def _update():
        rp = rprev_ref[...]
        q = qprev_ref[...]
        qs = rp + (q - rp)
        zqnew_ref[...] = zqprev_ref[...] + qs
        r = rp - qs
        rnew_ref[...] = r
        x2h_ref[...] = jnp.sum(r * r, axis=1, keepdims=True) * 0.5
        _init_minmax(rmin_ref, rarg_ref)

    _argmin_block(k, rnew_ref[...], w_ref, x2h_ref, y2h_ref, iota_ref,
                  rmin_ref, rarg_ref, idx_ref)


_scratch = [
    pltpu.VMEM((TM, 1), jnp.float32),
    pltpu.VMEM((1, K), jnp.float32),
    pltpu.VMEM((1, KB), jnp.float32),
    pltpu.VMEM((TM, 1), jnp.float32),
    pltpu.VMEM((TM, 1), jnp.int32),
]

_params = pltpu.CompilerParams(dimension_semantics=("arbitrary", "arbitrary"))

layer0_call = pl.pallas_call(
    layer0_body,
    grid=(T // TM, NKB),
    in_specs=[
        pl.BlockSpec((TM, D), lambda t, k: (t, 0)),
        pl.BlockSpec((KB, D), lambda t, k: (k, 0)),
    ],
    out_specs=pl.BlockSpec((TM,), lambda t, k: (t,)),
    out_shape=jax.ShapeDtypeStruct((T,), jnp.int32),
    scratch_shapes=_scratch,
    compiler_params=_params,
)

fused_call = pl.pallas_call(
    fused_body,
    grid=(T // TM, NKB),
    in_specs=[
        pl.BlockSpec((TM, D), lambda t, k: (t, 0)),
        pl.BlockSpec((TM, D), lambda t, k: (t, 0)),
        pl.BlockSpec((TM, D), lambda t, k: (t, 0)),
        pl.BlockSpec((KB, D), lambda t, k: (k, 0)),
    ],
    out_specs=[
        pl.BlockSpec((TM,), lambda t, k: (t,)),
        pl.BlockSpec((TM, D), lambda t, k: (t, 0)),
        pl.BlockSpec((TM, D), lambda t, k: (t, 0)),
    ],
    out_shape=[
        jax.ShapeDtypeStruct((T,), jnp.int32),
        jax.ShapeDtypeStruct((T, D), jnp.float32),
        jax.ShapeDtypeStruct((T, D), jnp.float32),
    ],
    scratch_shapes=_scratch,
    compiler_params=_params,
)


@functools.lru_cache(maxsize=1)
def _sc_gather():
    @functools.partial(
        pl.kernel,
        mesh=plsc.VectorSubcoreMesh(core_axis_name="c", subcore_axis_name="s"),
        out_type=jax.ShapeDtypeStruct((T, D), jnp.float32),
        scratch_types=[
            pltpu.VMEM((BPW,), jnp.int32),
            pltpu.VMEM((BPW, D), jnp.float32),
            pltpu.SemaphoreType.DMA,
        ],
    )
    def sc_gather(cb_hbm, gidx_hbm, out_hbm, idx_v, rows_v, sem):
        wid = lax.axis_index("s") * 2 + lax.axis_index("c")
        base = wid * BPW
        pltpu.sync_copy(gidx_hbm.at[pl.ds(base, BPW)], idx_v)
        pltpu.async_copy(cb_hbm.at[idx_v], rows_v, sem).wait()
        pltpu.sync_copy(rows_v, out_hbm.at[pl.ds(base, BPW)])

    return sc_gather


def kernel(z, codebooks):
    batch, dim, time = z.shape
    zt = jnp.transpose(z, (0, 2, 1))
    r0 = zt.reshape(T, D)
    cb_flat = codebooks.reshape(NL * K, D)
    gather = _sc_gather()

    idx = layer0_call(r0, codebooks[0])
    q = gather(cb_flat, idx)
    codes = [idx]
    r, zq = r0, jnp.zeros_like(r0)
    loss = jnp.zeros((), dtype=jnp.float32)
    for layer in range(1, NL):
        loss = loss + jnp.mean((r - q) ** 2)
        idx, r, zq = fused_call(r, q, zq, codebooks[layer])
        q = gather(cb_flat, idx + layer * K)
        codes.append(idx)

    loss = loss + jnp.mean((r - q) ** 2)
    qs = r + (q - r)
    zq = zq + qs

    z_q_out = jnp.transpose(zq.reshape(batch, time, dim), (0, 2, 1))
    all_codes = jnp.stack([c.reshape(batch, time) for c in codes], axis=0)
    return (z_q_out, all_codes, loss, loss, loss + loss)


# speedup vs baseline: 1.3638x; 1.0736x over previous
import functools

import jax
import jax.numpy as jnp
from jax import lax
from jax.experimental import pallas as pl
from jax.experimental.pallas import tpu as pltpu
from jax.experimental.pallas import tpu_sc as plsc

NL = 8
K = 8192
D = 256
T = 4096
TM = 1024
KB = 2048
NKB = K // KB

NW = 32
BPW = T // NW


def _argmin_block(k, r, w_ref, x2h_ref, y2h_ref, iota_ref, rmin_ref, rarg_ref,
                  idx_ref):
    w = w_ref[...]

    @pl.when(pl.program_id(0) == 0)
    def _y2():
        y2h_ref[pl.ds(k * KB, KB), :] = jnp.sum(w * w, axis=1,
                                                keepdims=True) * 0.5

    @pl.when((pl.program_id(0) == 0) & (k == 0))
    def _iota():
        iota_ref[...] = lax.broadcasted_iota(
            jnp.int32, (KB, 1), 0).astype(jnp.float32)

    s = lax.dot_general(w, r, (((1,), (1,)), ((), ())),
                        preferred_element_type=jnp.float32)
    d = (x2h_ref[...] + y2h_ref[pl.ds(k * KB, KB), :]) - s
    m = jnp.min(d, axis=0, keepdims=True)
    loc = jnp.min(jnp.where(d == m, iota_ref[...], float(KB)), axis=0,
                  keepdims=True)
    cand = loc.astype(jnp.int32) + k * KB
    better = m < rmin_ref[...]
    rarg_ref[...] = jnp.where(better, cand, rarg_ref[...])
    rmin_ref[...] = jnp.where(better, m, rmin_ref[...])

    @pl.when(k == NKB - 1)
    def _flush():
        idx_ref[...] = rarg_ref[...].reshape(1, 1, TM)


def _store_x2h(r, x2h_ref):
    x2col = jnp.sum(r * r, axis=1, keepdims=True) * 0.5
    x2h_ref[...] = jnp.transpose(x2col, (1, 0))


def _init_minmax(rmin_ref, rarg_ref):
    rmin_ref[...] = jnp.full((1, TM), jnp.inf, dtype=jnp.float32)
    rarg_ref[...] = jnp.zeros((1, TM), dtype=jnp.int32)


def layer0_body(r_ref, w_ref, idx_ref, x2h_ref, y2h_ref, iota_ref, rmin_ref,
                rarg_ref):
    k = pl.program_id(1)

    @pl.when(k == 0)
    def _init():
        _store_x2h(r_ref[...], x2h_ref)
        _init_minmax(rmin_ref, rarg_ref)

    _argmin_block(k, r_ref[...], w_ref, x2h_ref, y2h_ref, iota_ref,
                  rmin_ref, rarg_ref, idx_ref)


def fused_body(rprev_ref, qprev_ref, zqprev_ref, w_ref,
               idx_ref, rnew_ref, zqnew_ref,
               x2h_ref, y2h_ref, iota_ref, rmin_ref, rarg_ref):
    k = pl.program_id(1)

    @pl.when(k == 0)
    def _update():
        rp = rprev_ref[...]
        q = qprev_ref[...]
        qs = rp + (q - rp)
        zqnew_ref[...] = zqprev_ref[...] + qs
        r = rp - qs
        rnew_ref[...] = r
        _store_x2h(r, x2h_ref)
        _init_minmax(rmin_ref, rarg_ref)

    _argmin_block(k, rnew_ref[...], w_ref, x2h_ref, y2h_ref, iota_ref,
                  rmin_ref, rarg_ref, idx_ref)


_scratch = [
    pltpu.VMEM((1, TM), jnp.float32),
    pltpu.VMEM((K, 1), jnp.float32),
    pltpu.VMEM((KB, 1), jnp.float32),
    pltpu.VMEM((1, TM), jnp.float32),
    pltpu.VMEM((1, TM), jnp.int32),
]

_params = pltpu.CompilerParams(dimension_semantics=("arbitrary", "arbitrary"))

_IDX_SHAPE = jax.ShapeDtypeStruct((T // TM, 1, TM), jnp.int32)
_idx_spec = pl.BlockSpec((1, 1, TM), lambda t, k: (t, 0, 0))

layer0_call = pl.pallas_call(
    layer0_body,
    grid=(T // TM, NKB),
    in_specs=[
        pl.BlockSpec((TM, D), lambda t, k: (t, 0)),
        pl.BlockSpec((KB, D), lambda t, k: (k, 0)),
    ],
    out_specs=_idx_spec,
    out_shape=_IDX_SHAPE,
    scratch_shapes=_scratch,
    compiler_params=_params,
)

fused_call = pl.pallas_call(
    fused_body,
    grid=(T // TM, NKB),
    in_specs=[
        pl.BlockSpec((TM, D), lambda t, k: (t, 0)),
        pl.BlockSpec((TM, D), lambda t, k: (t, 0)),
        pl.BlockSpec((TM, D), lambda t, k: (t, 0)),
        pl.BlockSpec((KB, D), lambda t, k: (k, 0)),
    ],
    out_specs=[
        _idx_spec,
        pl.BlockSpec((TM, D), lambda t, k: (t, 0)),
        pl.BlockSpec((TM, D), lambda t, k: (t, 0)),
    ],
    out_shape=[
        _IDX_SHAPE,
        jax.ShapeDtypeStruct((T, D), jnp.float32),
        jax.ShapeDtypeStruct((T, D), jnp.float32),
    ],
    scratch_shapes=_scratch,
    compiler_params=_params,
)


@functools.lru_cache(maxsize=1)
def _sc_gather():
    @functools.partial(
        pl.kernel,
        mesh=plsc.VectorSubcoreMesh(core_axis_name="c", subcore_axis_name="s"),
        out_type=jax.ShapeDtypeStruct((T, D), jnp.float32),
        scratch_types=[
            pltpu.VMEM((BPW,), jnp.int32),
            pltpu.VMEM((BPW, D), jnp.float32),
            pltpu.SemaphoreType.DMA,
        ],
    )
    def sc_gather(cb_hbm, gidx_hbm, out_hbm, idx_v, rows_v, sem):
        wid = lax.axis_index("s") * 2 + lax.axis_index("c")
        base = wid * BPW
        pltpu.sync_copy(gidx_hbm.at[pl.ds(base, BPW)], idx_v)
        pltpu.async_copy(cb_hbm.at[idx_v], rows_v, sem).wait()
        pltpu.sync_copy(rows_v, out_hbm.at[pl.ds(base, BPW)])

    return sc_gather


def kernel(z, codebooks):
    batch, dim, time = z.shape
    zt = jnp.transpose(z, (0, 2, 1))
    r0 = zt.reshape(T, D)
    cb_flat = codebooks.reshape(NL * K, D)
    gather = _sc_gather()

    idx = layer0_call(r0, codebooks[0]).reshape(T)
    q = gather(cb_flat, idx)
    codes = [idx]
    r, zq = r0, jnp.zeros_like(r0)
    loss = jnp.zeros((), dtype=jnp.float32)
    for layer in range(1, NL):
        loss = loss + jnp.mean((r - q) ** 2)
        idx, r, zq = fused_call(r, q, zq, codebooks[layer])
        idx = idx.reshape(T)
        q = gather(cb_flat, idx + layer * K)
        codes.append(idx)

    loss = loss + jnp.mean((r - q) ** 2)
    qs = r + (q - r)
    zq = zq + qs

    z_q_out = jnp.transpose(zq.reshape(batch, time, dim), (0, 2, 1))
    all_codes = jnp.stack([c.reshape(batch, time) for c in codes], axis=0)
    return (z_q_out, all_codes, loss, loss, loss + loss)


# speedup vs baseline: 1.6471x; 1.2077x over previous
import functools

import jax
import jax.numpy as jnp
from jax import lax
from jax.experimental import pallas as pl
from jax.experimental.pallas import tpu as pltpu
from jax.experimental.pallas import tpu_sc as plsc

NL = 8
K = 8192
D = 256
T = 4096
TM = 1024
KB = 2048
NKB = K // KB

NW = 32
BPW = T // NW


def _argmin_block(k, r, w_ref, x2h_ref, y2h_ref, macc_ref, iacc_ref, idx_ref):
    w = w_ref[...]

    @pl.when(pl.program_id(0) == 0)
    def _y2():
        y2h_ref[pl.ds(k * KB, KB), :] = jnp.sum(w * w, axis=1,
                                                keepdims=True) * 0.5

    @pl.when(k == 0)
    def _init():
        macc_ref[...] = jnp.full((8, TM), jnp.inf, dtype=jnp.float32)
        iacc_ref[...] = jnp.zeros((8, TM), dtype=jnp.float32)

    s = lax.dot_general(w, r, (((1,), (1,)), ((), ())),
                        preferred_element_type=jnp.float32)
    x2h = x2h_ref[...]
    macc = macc_ref[...]
    iacc = iacc_ref[...]
    base = lax.convert_element_type(k * (KB // 8), jnp.float32)
    for i in range(KB // 8):
        y2i = y2h_ref[pl.ds(k * KB + i * 8, 8), :]
        di = (x2h + y2i) - s[i * 8:(i + 1) * 8, :]
        mask = di < macc
        iacc = jnp.where(mask, base + float(i), iacc)
        macc = jnp.where(mask, di, macc)
    macc_ref[...] = macc
    iacc_ref[...] = iacc

    @pl.when(k == NKB - 1)
    def _flush():
        subl = lax.broadcasted_iota(jnp.int32, (8, TM), 0).astype(jnp.float32)
        kv = iacc * 8.0 + subl
        m = jnp.min(macc, axis=0, keepdims=True)
        loc = jnp.min(jnp.where(macc == m, kv, float(K)), axis=0,
                      keepdims=True)
        idx_ref[...] = loc.astype(jnp.int32).reshape(1, 1, TM)


def _store_x2h(r, x2h_ref):
    x2col = jnp.sum(r * r, axis=1, keepdims=True) * 0.5
    x2h_ref[...] = jnp.transpose(x2col, (1, 0))


def layer0_body(r_ref, w_ref, idx_ref, x2h_ref, y2h_ref, macc_ref, iacc_ref):
    k = pl.program_id(1)

    @pl.when(k == 0)
    def _init():
        _store_x2h(r_ref[...], x2h_ref)

    _argmin_block(k, r_ref[...], w_ref, x2h_ref, y2h_ref, macc_ref, iacc_ref,
                  idx_ref)


def fused_body(rprev_ref, qprev_ref, zqprev_ref, w_ref,
               idx_ref, rnew_ref, zqnew_ref,
               x2h_ref, y2h_ref, macc_ref, iacc_ref):
    k = pl.program_id(1)

    @pl.when(k == 0)
    def _update():
        rp = rprev_ref[...]
        q = qprev_ref[...]
        qs = rp + (q - rp)
        zqnew_ref[...] = zqprev_ref[...] + qs
        r = rp - qs
        rnew_ref[...] = r
        _store_x2h(r, x2h_ref)

    _argmin_block(k, rnew_ref[...], w_ref, x2h_ref, y2h_ref, macc_ref,
                  iacc_ref, idx_ref)


_scratch = [
    pltpu.VMEM((1, TM), jnp.float32),
    pltpu.VMEM((K, 1), jnp.float32),
    pltpu.VMEM((8, TM), jnp.float32),
    pltpu.VMEM((8, TM), jnp.float32),
]

_params = pltpu.CompilerParams(dimension_semantics=("arbitrary", "arbitrary"))

_IDX_SHAPE = jax.ShapeDtypeStruct((T // TM, 1, TM), jnp.int32)
_idx_spec = pl.BlockSpec((1, 1, TM), lambda t, k: (t, 0, 0))

layer0_call = pl.pallas_call(
    layer0_body,
    grid=(T // TM, NKB),
    in_specs=[
        pl.BlockSpec((TM, D), lambda t, k: (t, 0)),
        pl.BlockSpec((KB, D), lambda t, k: (k, 0)),
    ],
    out_specs=_idx_spec,
    out_shape=_IDX_SHAPE,
    scratch_shapes=_scratch,
    compiler_params=_params,
)

fused_call = pl.pallas_call(
    fused_body,
    grid=(T // TM, NKB),
    in_specs=[
        pl.BlockSpec((TM, D), lambda t, k: (t, 0)),
        pl.BlockSpec((TM, D), lambda t, k: (t, 0)),
        pl.BlockSpec((TM, D), lambda t, k: (t, 0)),
        pl.BlockSpec((KB, D), lambda t, k: (k, 0)),
    ],
    out_specs=[
        _idx_spec,
        pl.BlockSpec((TM, D), lambda t, k: (t, 0)),
        pl.BlockSpec((TM, D), lambda t, k: (t, 0)),
    ],
    out_shape=[
        _IDX_SHAPE,
        jax.ShapeDtypeStruct((T, D), jnp.float32),
        jax.ShapeDtypeStruct((T, D), jnp.float32),
    ],
    scratch_shapes=_scratch,
    compiler_params=_params,
)


@functools.lru_cache(maxsize=1)
def _sc_gather():
    @functools.partial(
        pl.kernel,
        mesh=plsc.VectorSubcoreMesh(core_axis_name="c", subcore_axis_name="s"),
        out_type=jax.ShapeDtypeStruct((T, D), jnp.float32),
        scratch_types=[
            pltpu.VMEM((BPW,), jnp.int32),
            pltpu.VMEM((BPW, D), jnp.float32),
            pltpu.SemaphoreType.DMA,
        ],
    )
    def sc_gather(cb_hbm, gidx_hbm, out_hbm, idx_v, rows_v, sem):
        wid = lax.axis_index("s") * 2 + lax.axis_index("c")
        base = wid * BPW
        pltpu.sync_copy(gidx_hbm.at[pl.ds(base, BPW)], idx_v)
        pltpu.async_copy(cb_hbm.at[idx_v], rows_v, sem).wait()
        pltpu.sync_copy(rows_v, out_hbm.at[pl.ds(base, BPW)])

    return sc_gather


def kernel(z, codebooks):
    batch, dim, time = z.shape
    zt = jnp.transpose(z, (0, 2, 1))
    r0 = zt.reshape(T, D)
    cb_flat = codebooks.reshape(NL * K, D)
    gather = _sc_gather()

    idx = layer0_call(r0, codebooks[0]).reshape(T)
    q = gather(cb_flat, idx)
    codes = [idx]
    r, zq = r0, jnp.zeros_like(r0)
    loss = jnp.zeros((), dtype=jnp.float32)
    for layer in range(1, NL):
        loss = loss + jnp.mean((r - q) ** 2)
        idx, r, zq = fused_call(r, q, zq, codebooks[layer])
        idx = idx.reshape(T)
        q = gather(cb_flat, idx + layer * K)
        codes.append(idx)

    loss = loss + jnp.mean((r - q) ** 2)
    qs = r + (q - r)
    zq = zq + qs

    z_q_out = jnp.transpose(zq.reshape(batch, time, dim), (0, 2, 1))
    all_codes = jnp.stack([c.reshape(batch, time) for c in codes], axis=0)
    return (z_q_out, all_codes, loss, loss, loss + loss)


# speedup vs baseline: 1.7828x; 1.0824x over previous
import functools

import jax
import jax.numpy as jnp
from jax import lax
from jax.experimental import pallas as pl
from jax.experimental.pallas import tpu as pltpu
from jax.experimental.pallas import tpu_sc as plsc

NL = 8
K = 8192
D = 256
T = 4096
TM = 1024
KB = 2048
NKB = K // KB

NW = 32
BPW = T // NW


def _argmin_block(k, r, w_ref, x2h_ref, y2h_ref, macc_ref, iacc_ref, idx_ref):
    w = w_ref[...]

    @pl.when(pl.program_id(0) == 0)
    def _y2():
        y2h_ref[pl.ds(k * KB, KB), :] = jnp.sum(w * w, axis=1,
                                                keepdims=True) * 0.5

    @pl.when(k == 0)
    def _init():
        macc_ref[...] = jnp.full((8, TM), jnp.inf, dtype=jnp.float32)
        iacc_ref[...] = jnp.zeros((8, TM), dtype=jnp.float32)

    s = lax.dot_general(w, r, (((1,), (1,)), ((), ())),
                        preferred_element_type=jnp.float32)
    x2h = x2h_ref[...]
    macc = macc_ref[...]
    iacc = iacc_ref[...]
    base = lax.convert_element_type(k * (KB // 8), jnp.float32)
    for i in range(KB // 8):
        y2i = y2h_ref[pl.ds(k * KB + i * 8, 8), :]
        di = (x2h + y2i) - s[i * 8:(i + 1) * 8, :]
        mask = di < macc
        iacc = jnp.where(mask, base + float(i), iacc)
        macc = jnp.where(mask, di, macc)
    macc_ref[...] = macc
    iacc_ref[...] = iacc

    @pl.when(k == NKB - 1)
    def _flush():
        subl = lax.broadcasted_iota(jnp.int32, (8, TM), 0).astype(jnp.float32)
        kv = iacc * 8.0 + subl
        m = jnp.min(macc, axis=0, keepdims=True)
        loc = jnp.min(jnp.where(macc == m, kv, float(K)), axis=0,
                      keepdims=True)
        idx_ref[...] = loc.astype(jnp.int32).reshape(1, 1, TM)


def _store_x2h(r, x2h_ref):
    x2col = jnp.sum(r * r, axis=1, keepdims=True) * 0.5
    x2h_ref[...] = jnp.transpose(x2col, (1, 0))


def layer0_body(r_ref, w_ref, idx_ref, x2h_ref, y2h_ref, macc_ref, iacc_ref):
    k = pl.program_id(1)

    @pl.when(k == 0)
    def _init():
        _store_x2h(r_ref[...], x2h_ref)

    _argmin_block(k, r_ref[...], w_ref, x2h_ref, y2h_ref, macc_ref, iacc_ref,
                  idx_ref)


def fused_body(rprev_ref, qprev_ref, zqprev_ref, w_ref,
               idx_ref, rnew_ref, zqnew_ref, lsum_ref,
               x2h_ref, y2h_ref, macc_ref, iacc_ref, lacc_ref):
    t = pl.program_id(0)
    k = pl.program_id(1)

    @pl.when(k == 0)
    def _update():
        rp = rprev_ref[...]
        q = qprev_ref[...]
        diff = rp - q
        part = jnp.sum(diff * diff)

        @pl.when(t == 0)
        def _l0():
            lacc_ref[0] = part

        @pl.when(t > 0)
        def _ln():
            lacc_ref[0] = lacc_ref[0] + part

        qs = rp + (q - rp)
        zqnew_ref[...] = zqprev_ref[...] + qs
        r = rp - qs
        rnew_ref[...] = r
        _store_x2h(r, x2h_ref)

    @pl.when((t == T // TM - 1) & (k == NKB - 1))
    def _lout():
        lsum_ref[0, 0] = lacc_ref[0]

    _argmin_block(k, rnew_ref[...], w_ref, x2h_ref, y2h_ref, macc_ref,
                  iacc_ref, idx_ref)


_scratch = [
    pltpu.VMEM((1, TM), jnp.float32),
    pltpu.VMEM((K, 1), jnp.float32),
    pltpu.VMEM((8, TM), jnp.float32),
    pltpu.VMEM((8, TM), jnp.float32),
]

_params = pltpu.CompilerParams(dimension_semantics=("arbitrary", "arbitrary"))

_IDX_SHAPE = jax.ShapeDtypeStruct((T // TM, 1, TM), jnp.int32)
_idx_spec = pl.BlockSpec((1, 1, TM), lambda t, k: (t, 0, 0))

layer0_call = pl.pallas_call(
    layer0_body,
    grid=(T // TM, NKB),
    in_specs=[
        pl.BlockSpec((TM, D), lambda t, k: (t, 0)),
        pl.BlockSpec((KB, D), lambda t, k: (k, 0)),
    ],
    out_specs=_idx_spec,
    out_shape=_IDX_SHAPE,
    scratch_shapes=_scratch,
    compiler_params=_params,
)

fused_call = pl.pallas_call(
    fused_body,
    grid=(T // TM, NKB),
    in_specs=[
        pl.BlockSpec((TM, D), lambda t, k: (t, 0)),
        pl.BlockSpec((TM, D), lambda t, k: (t, 0)),
        pl.BlockSpec((TM, D), lambda t, k: (t, 0)),
        pl.BlockSpec((KB, D), lambda t, k: (k, 0)),
    ],
    out_specs=[
        _idx_spec,
        pl.BlockSpec((TM, D), lambda t, k: (t, 0)),
        pl.BlockSpec((TM, D), lambda t, k: (t, 0)),
        pl.BlockSpec(memory_space=pltpu.SMEM),
    ],
    out_shape=[
        _IDX_SHAPE,
        jax.ShapeDtypeStruct((T, D), jnp.float32),
        jax.ShapeDtypeStruct((T, D), jnp.float32),
        jax.ShapeDtypeStruct((1, 1), jnp.float32),
    ],
    scratch_shapes=_scratch + [pltpu.SMEM((1,), jnp.float32)],
    compiler_params=_params,
)


@functools.lru_cache(maxsize=None)
def _sc_gather(off):
    @functools.partial(
        pl.kernel,
        mesh=plsc.VectorSubcoreMesh(core_axis_name="c", subcore_axis_name="s"),
        out_type=jax.ShapeDtypeStruct((T, D), jnp.float32),
        scratch_types=[
            pltpu.VMEM((BPW,), jnp.int32),
            pltpu.VMEM((BPW, D), jnp.float32),
            pltpu.SemaphoreType.DMA,
        ],
    )
    def sc_gather(cb_hbm, gidx_hbm, out_hbm, idx_v, rows_v, sem):
        wid = lax.axis_index("s") * 2 + lax.axis_index("c")
        base = wid * BPW
        pltpu.sync_copy(gidx_hbm.at[pl.ds(base, BPW)], idx_v)
        if off:
            for j in range(BPW // 16):
                sl = pl.ds(j * 16, 16)
                idx_v[sl] = idx_v[sl] + off
        pltpu.async_copy(cb_hbm.at[idx_v], rows_v, sem).wait()
        pltpu.sync_copy(rows_v, out_hbm.at[pl.ds(base, BPW)])

    return sc_gather


def kernel(z, codebooks):
    batch, dim, time = z.shape
    zt = jnp.transpose(z, (0, 2, 1))
    r0 = zt.reshape(T, D)
    cb_flat = codebooks.reshape(NL * K, D)
    idx = layer0_call(r0, codebooks[0]).reshape(T)
    q = _sc_gather(0)(cb_flat, idx)
    codes = [idx]
    r, zq = r0, jnp.zeros_like(r0)
    loss = jnp.zeros((), dtype=jnp.float32)
    inv_n = jnp.float32(1.0 / (T * D))
    for layer in range(1, NL):
        idx, r, zq, lsum = fused_call(r, q, zq, codebooks[layer])
        loss = loss + lsum[0, 0] * inv_n
        idx = idx.reshape(T)
        q = _sc_gather(layer * K)(cb_flat, idx)
        codes.append(idx)

    loss = loss + jnp.mean((r - q) ** 2)
    qs = r + (q - r)
    zq = zq + qs

    z_q_out = jnp.transpose(zq.reshape(batch, time, dim), (0, 2, 1))
    all_codes = jnp.stack([c.reshape(batch, time) for c in codes], axis=0)
    return (z_q_out, all_codes, loss, loss, loss + loss)


# speedup vs baseline: 1.8694x; 1.0486x over previous
import functools

import jax
import jax.numpy as jnp
from jax import lax
from jax.experimental import pallas as pl
from jax.experimental.pallas import tpu as pltpu
from jax.experimental.pallas import tpu_sc as plsc

NL = 8
K = 8192
D = 256
T = 4096
TM = 2048
KB = 2048
NKB = K // KB

NW = 32
BPW = T // NW


def _argmin_block(k, r, w_ref, x2h_ref, y2h_ref, macc_ref, iacc_ref, idx_ref):
    w = w_ref[...]

    @pl.when(pl.program_id(0) == 0)
    def _y2():
        y2h_ref[pl.ds(k * KB, KB), :] = jnp.sum(w * w, axis=1,
                                                keepdims=True) * 0.5

    @pl.when(k == 0)
    def _init():
        macc_ref[...] = jnp.full((8, TM), jnp.inf, dtype=jnp.float32)
        iacc_ref[...] = jnp.zeros((8, TM), dtype=jnp.float32)

    s = lax.dot_general(w, r, (((1,), (1,)), ((), ())),
                        preferred_element_type=jnp.float32)
    x2h = x2h_ref[...]
    macc = macc_ref[...]
    iacc = iacc_ref[...]
    base = lax.convert_element_type(k * (KB // 8), jnp.float32)
    for i in range(KB // 8):
        y2i = y2h_ref[pl.ds(k * KB + i * 8, 8), :]
        di = (x2h + y2i) - s[i * 8:(i + 1) * 8, :]
        mask = di < macc
        iacc = jnp.where(mask, base + float(i), iacc)
        macc = jnp.where(mask, di, macc)
    macc_ref[...] = macc
    iacc_ref[...] = iacc

    @pl.when(k == NKB - 1)
    def _flush():
        subl = lax.broadcasted_iota(jnp.int32, (8, TM), 0).astype(jnp.float32)
        kv = iacc * 8.0 + subl
        m = jnp.min(macc, axis=0, keepdims=True)
        loc = jnp.min(jnp.where(macc == m, kv, float(K)), axis=0,
                      keepdims=True)
        idx_ref[...] = loc.astype(jnp.int32).reshape(1, 1, TM)


def _store_x2h(r, x2h_ref):
    x2col = jnp.sum(r * r, axis=1, keepdims=True) * 0.5
    x2h_ref[...] = jnp.transpose(x2col, (1, 0))


def layer0_body(r_ref, w_ref, idx_ref, x2h_ref, y2h_ref, macc_ref, iacc_ref):
    k = pl.program_id(1)

    @pl.when(k == 0)
    def _init():
        _store_x2h(r_ref[...], x2h_ref)

    _argmin_block(k, r_ref[...], w_ref, x2h_ref, y2h_ref, macc_ref, iacc_ref,
                  idx_ref)


def fused_body(rprev_ref, qprev_ref, zqprev_ref, w_ref,
               idx_ref, rnew_ref, zqnew_ref, lsum_ref,
               x2h_ref, y2h_ref, macc_ref, iacc_ref, lacc_ref):
    t = pl.program_id(0)
    k = pl.program_id(1)

    @pl.when(k == 0)
    def _update():
        rp = rprev_ref[...]
        q = qprev_ref[...]
        diff = rp - q
        part = jnp.sum(diff * diff)

        @pl.when(t == 0)
        def _l0():
            lacc_ref[0] = part

        @pl.when(t > 0)
        def _ln():
            lacc_ref[0] = lacc_ref[0] + part

        qs = rp + (q - rp)
        zqnew_ref[...] = zqprev_ref[...] + qs
        r = rp - qs
        rnew_ref[...] = r
        _store_x2h(r, x2h_ref)

    @pl.when((t == T // TM - 1) & (k == NKB - 1))
    def _lout():
        lsum_ref[0, 0] = lacc_ref[0]

    _argmin_block(k, rnew_ref[...], w_ref, x2h_ref, y2h_ref, macc_ref,
                  iacc_ref, idx_ref)


_scratch = [
    pltpu.VMEM((1, TM), jnp.float32),
    pltpu.VMEM((K, 1), jnp.float32),
    pltpu.VMEM((8, TM), jnp.float32),
    pltpu.VMEM((8, TM), jnp.float32),
]

_params = pltpu.CompilerParams(dimension_semantics=("arbitrary", "arbitrary"))

_IDX_SHAPE = jax.ShapeDtypeStruct((T // TM, 1, TM), jnp.int32)
_idx_spec = pl.BlockSpec((1, 1, TM), lambda t, k: (t, 0, 0))

layer0_call = pl.pallas_call(
    layer0_body,
    grid=(T // TM, NKB),
    in_specs=[
        pl.BlockSpec((TM, D), lambda t, k: (t, 0)),
        pl.BlockSpec((KB, D), lambda t, k: (k, 0)),
    ],
    out_specs=_idx_spec,
    out_shape=_IDX_SHAPE,
    scratch_shapes=_scratch,
    compiler_params=_params,
)

fused_call = pl.pallas_call(
    fused_body,
    grid=(T // TM, NKB),
    in_specs=[
        pl.BlockSpec((TM, D), lambda t, k: (t, 0)),
        pl.BlockSpec((TM, D), lambda t, k: (t, 0)),
        pl.BlockSpec((TM, D), lambda t, k: (t, 0)),
        pl.BlockSpec((KB, D), lambda t, k: (k, 0)),
    ],
    out_specs=[
        _idx_spec,
        pl.BlockSpec((TM, D), lambda t, k: (t, 0)),
        pl.BlockSpec((TM, D), lambda t, k: (t, 0)),
        pl.BlockSpec(memory_space=pltpu.SMEM),
    ],
    out_shape=[
        _IDX_SHAPE,
        jax.ShapeDtypeStruct((T, D), jnp.float32),
        jax.ShapeDtypeStruct((T, D), jnp.float32),
        jax.ShapeDtypeStruct((1, 1), jnp.float32),
    ],
    scratch_shapes=_scratch + [pltpu.SMEM((1,), jnp.float32)],
    compiler_params=_params,
)


@functools.lru_cache(maxsize=None)
def _sc_gather(off):
    @functools.partial(
        pl.kernel,
        mesh=plsc.VectorSubcoreMesh(core_axis_name="c", subcore_axis_name="s"),
        out_type=jax.ShapeDtypeStruct((T, D), jnp.float32),
        scratch_types=[
            pltpu.VMEM((BPW,), jnp.int32),
            pltpu.VMEM((BPW, D), jnp.float32),
            pltpu.SemaphoreType.DMA,
        ],
    )
    def sc_gather(cb_hbm, gidx_hbm, out_hbm, idx_v, rows_v, sem):
        wid = lax.axis_index("s") * 2 + lax.axis_index("c")
        base = wid * BPW
        pltpu.sync_copy(gidx_hbm.at[pl.ds(base, BPW)], idx_v)
        if off:
            for j in range(BPW // 16):
                sl = pl.ds(j * 16, 16)
                idx_v[sl] = idx_v[sl] + off
        pltpu.async_copy(cb_hbm.at[idx_v], rows_v, sem).wait()
        pltpu.sync_copy(rows_v, out_hbm.at[pl.ds(base, BPW)])

    return sc_gather


def kernel(z, codebooks):
    batch, dim, time = z.shape
    zt = jnp.transpose(z, (0, 2, 1))
    r0 = zt.reshape(T, D)
    cb_flat = codebooks.reshape(NL * K, D)
    idx = layer0_call(r0, codebooks[0]).reshape(T)
    q = _sc_gather(0)(cb_flat, idx)
    codes = [idx]
    r, zq = r0, jnp.zeros_like(r0)
    loss = jnp.zeros((), dtype=jnp.float32)
    inv_n = jnp.float32(1.0 / (T * D))
    for layer in range(1, NL):
        idx, r, zq, lsum = fused_call(r, q, zq, codebooks[layer])
        loss = loss + lsum[0, 0] * inv_n
        idx = idx.reshape(T)
        q = _sc_gather(layer * K)(cb_flat, idx)
        codes.append(idx)

    loss = loss + jnp.mean((r - q) ** 2)
    qs = r + (q - r)
    zq = zq + qs

    z_q_out = jnp.transpose(zq.reshape(batch, time, dim), (0, 2, 1))
    all_codes = jnp.stack([c.reshape(batch, time) for c in codes], axis=0)
    return (z_q_out, all_codes, loss, loss, loss + loss)


# speedup vs baseline: 1.9279x; 1.0313x over previous
import functools

import jax
import jax.numpy as jnp
from jax import lax
from jax.experimental import pallas as pl
from jax.experimental.pallas import tpu as pltpu
from jax.experimental.pallas import tpu_sc as plsc

NL = 8
K = 8192
D = 256
T = 4096
TM = 2048
KB = 4096
NKB = K // KB

NW = 32
BPW = T // NW


def _argmin_block(k, r, w_ref, x2h_ref, y2h_ref, macc_ref, iacc_ref, idx_ref):
    w = w_ref[...]

    @pl.when(pl.program_id(0) == 0)
    def _y2():
        y2h_ref[pl.ds(k * KB, KB), :] = jnp.sum(w * w, axis=1,
                                                keepdims=True) * 0.5

    @pl.when(k == 0)
    def _init():
        macc_ref[...] = jnp.full((8, TM), jnp.inf, dtype=jnp.float32)
        iacc_ref[...] = jnp.zeros((8, TM), dtype=jnp.float32)

    s = lax.dot_general(w, r, (((1,), (1,)), ((), ())),
                        preferred_element_type=jnp.float32)
    x2h = x2h_ref[...]
    macc = macc_ref[...]
    iacc = iacc_ref[...]
    base = lax.convert_element_type(k * (KB // 8), jnp.float32)
    for i in range(KB // 8):
        y2i = y2h_ref[pl.ds(k * KB + i * 8, 8), :]
        di = (x2h + y2i) - s[i * 8:(i + 1) * 8, :]
        mask = di < macc
        iacc = jnp.where(mask, base + float(i), iacc)
        macc = jnp.where(mask, di, macc)
    macc_ref[...] = macc
    iacc_ref[...] = iacc

    @pl.when(k == NKB - 1)
    def _flush():
        subl = lax.broadcasted_iota(jnp.int32, (8, TM), 0).astype(jnp.float32)
        kv = iacc * 8.0 + subl
        m = jnp.min(macc, axis=0, keepdims=True)
        loc = jnp.min(jnp.where(macc == m, kv, float(K)), axis=0,
                      keepdims=True)
        idx_ref[...] = loc.astype(jnp.int32).reshape(1, 1, TM)


def _store_x2h(r, x2h_ref):
    x2col = jnp.sum(r * r, axis=1, keepdims=True) * 0.5
    x2h_ref[...] = jnp.transpose(x2col, (1, 0))


def layer0_body(r_ref, w_ref, idx_ref, x2h_ref, y2h_ref, macc_ref, iacc_ref):
    k = pl.program_id(1)

    @pl.when(k == 0)
    def _init():
        _store_x2h(r_ref[...], x2h_ref)

    _argmin_block(k, r_ref[...], w_ref, x2h_ref, y2h_ref, macc_ref, iacc_ref,
                  idx_ref)


def fused_body(rprev_ref, qprev_ref, zqprev_ref, w_ref,
               idx_ref, rnew_ref, zqnew_ref, lsum_ref,
               x2h_ref, y2h_ref, macc_ref, iacc_ref, lacc_ref):
    t = pl.program_id(0)
    k = pl.program_id(1)

    @pl.when(k == 0)
    def _update():
        rp = rprev_ref[...]
        q = qprev_ref[...]
        diff = rp - q
        part = jnp.sum(diff * diff)

        @pl.when(t == 0)
        def _l0():
            lacc_ref[0] = part

        @pl.when(t > 0)
        def _ln():
            lacc_ref[0] = lacc_ref[0] + part

        qs = rp + (q - rp)
        zqnew_ref[...] = zqprev_ref[...] + qs
        r = rp - qs
        rnew_ref[...] = r
        _store_x2h(r, x2h_ref)

    @pl.when((t == T // TM - 1) & (k == NKB - 1))
    def _lout():
        lsum_ref[0, 0] = lacc_ref[0]

    _argmin_block(k, rnew_ref[...], w_ref, x2h_ref, y2h_ref, macc_ref,
                  iacc_ref, idx_ref)


_scratch = [
    pltpu.VMEM((1, TM), jnp.float32),
    pltpu.VMEM((K, 1), jnp.float32),
    pltpu.VMEM((8, TM), jnp.float32),
    pltpu.VMEM((8, TM), jnp.float32),
]

_params = pltpu.CompilerParams(dimension_semantics=("arbitrary", "arbitrary"))

_IDX_SHAPE = jax.ShapeDtypeStruct((T // TM, 1, TM), jnp.int32)
_idx_spec = pl.BlockSpec((1, 1, TM), lambda t, k: (t, 0, 0))

layer0_call = pl.pallas_call(
    layer0_body,
    grid=(T // TM, NKB),
    in_specs=[
        pl.BlockSpec((TM, D), lambda t, k: (t, 0)),
        pl.BlockSpec((KB, D), lambda t, k: (k, 0)),
    ],
    out_specs=_idx_spec,
    out_shape=_IDX_SHAPE,
    scratch_shapes=_scratch,
    compiler_params=_params,
)

fused_call = pl.pallas_call(
    fused_body,
    grid=(T // TM, NKB),
    in_specs=[
        pl.BlockSpec((TM, D), lambda t, k: (t, 0)),
        pl.BlockSpec((TM, D), lambda t, k: (t, 0)),
        pl.BlockSpec((TM, D), lambda t, k: (t, 0)),
        pl.BlockSpec((KB, D), lambda t, k: (k, 0)),
    ],
    out_specs=[
        _idx_spec,
        pl.BlockSpec((TM, D), lambda t, k: (t, 0)),
        pl.BlockSpec((TM, D), lambda t, k: (t, 0)),
        pl.BlockSpec(memory_space=pltpu.SMEM),
    ],
    out_shape=[
        _IDX_SHAPE,
        jax.ShapeDtypeStruct((T, D), jnp.float32),
        jax.ShapeDtypeStruct((T, D), jnp.float32),
        jax.ShapeDtypeStruct((1, 1), jnp.float32),
    ],
    scratch_shapes=_scratch + [pltpu.SMEM((1,), jnp.float32)],
    compiler_params=_params,
)


@functools.lru_cache(maxsize=None)
def _sc_gather(off):
    @functools.partial(
        pl.kernel,
        mesh=plsc.VectorSubcoreMesh(core_axis_name="c", subcore_axis_name="s"),
        out_type=jax.ShapeDtypeStruct((T, D), jnp.float32),
        scratch_types=[
            pltpu.VMEM((BPW,), jnp.int32),
            pltpu.VMEM((BPW, D), jnp.float32),
            pltpu.SemaphoreType.DMA,
        ],
    )
    def sc_gather(cb_hbm, gidx_hbm, out_hbm, idx_v, rows_v, sem):
        wid = lax.axis_index("s") * 2 + lax.axis_index("c")
        base = wid * BPW
        pltpu.sync_copy(gidx_hbm.at[pl.ds(base, BPW)], idx_v)
        if off:
            for j in range(BPW // 16):
                sl = pl.ds(j * 16, 16)
                idx_v[sl] = idx_v[sl] + off
        pltpu.async_copy(cb_hbm.at[idx_v], rows_v, sem).wait()
        pltpu.sync_copy(rows_v, out_hbm.at[pl.ds(base, BPW)])

    return sc_gather


def kernel(z, codebooks):
    batch, dim, time = z.shape
    zt = jnp.transpose(z, (0, 2, 1))
    r0 = zt.reshape(T, D)
    cb_flat = codebooks.reshape(NL * K, D)
    idx = layer0_call(r0, codebooks[0]).reshape(T)
    q = _sc_gather(0)(cb_flat, idx)
    codes = [idx]
    r, zq = r0, jnp.zeros_like(r0)
    loss = jnp.zeros((), dtype=jnp.float32)
    inv_n = jnp.float32(1.0 / (T * D))
    for layer in range(1, NL):
        idx, r, zq, lsum = fused_call(r, q, zq, codebooks[layer])
        loss = loss + lsum[0, 0] * inv_n
        idx = idx.reshape(T)
        q = _sc_gather(layer * K)(cb_flat, idx)
        codes.append(idx)

    loss = loss + jnp.mean((r - q) ** 2)
    qs = r + (q - r)
    zq = zq + qs

    z_q_out = jnp.transpose(zq.reshape(batch, time, dim), (0, 2, 1))
    all_codes = jnp.stack([c.reshape(batch, time) for c in codes], axis=0)
    return (z_q_out, all_codes, loss, loss, loss + loss)


# speedup vs baseline: 2.0158x; 1.0456x over previous
import functools

import jax
import jax.numpy as jnp
from jax import lax
from jax.experimental import pallas as pl
from jax.experimental.pallas import tpu as pltpu
from jax.experimental.pallas import tpu_sc as plsc

NL = 8
K = 8192
D = 256
T = 4096
TM = 1024
KB = 8192
NKB = K // KB

NW = 32
BPW = T // NW


def _argmin_block(k, r, w_ref, x2h_ref, y2h_ref, macc_ref, iacc_ref, idx_ref):
    w = w_ref[...]

    @pl.when(pl.program_id(0) == 0)
    def _y2():
        y2h_ref[pl.ds(k * KB, KB), :] = jnp.sum(w * w, axis=1,
                                                keepdims=True) * 0.5

    @pl.when(k == 0)
    def _init():
        macc_ref[...] = jnp.full((8, TM), jnp.inf, dtype=jnp.float32)
        iacc_ref[...] = jnp.zeros((8, TM), dtype=jnp.float32)

    s = lax.dot_general(w, r, (((1,), (1,)), ((), ())),
                        preferred_element_type=jnp.float32)
    x2h = x2h_ref[...]
    macc = macc_ref[...]
    iacc = iacc_ref[...]
    base = lax.convert_element_type(k * (KB // 8), jnp.float32)
    for i in range(KB // 8):
        y2i = y2h_ref[pl.ds(k * KB + i * 8, 8), :]
        di = (x2h + y2i) - s[i * 8:(i + 1) * 8, :]
        mask = di < macc
        iacc = jnp.where(mask, base + float(i), iacc)
        macc = jnp.where(mask, di, macc)
    macc_ref[...] = macc
    iacc_ref[...] = iacc

    @pl.when(k == NKB - 1)
    def _flush():
        subl = lax.broadcasted_iota(jnp.int32, (8, TM), 0).astype(jnp.float32)
        kv = iacc * 8.0 + subl
        m = jnp.min(macc, axis=0, keepdims=True)
        loc = jnp.min(jnp.where(macc == m, kv, float(K)), axis=0,
                      keepdims=True)
        idx_ref[...] = loc.astype(jnp.int32).reshape(1, 1, TM)


def _store_x2h(r, x2h_ref):
    x2col = jnp.sum(r * r, axis=1, keepdims=True) * 0.5
    x2h_ref[...] = jnp.transpose(x2col, (1, 0))


def layer0_body(r_ref, w_ref, idx_ref, x2h_ref, y2h_ref, macc_ref, iacc_ref):
    k = pl.program_id(1)

    @pl.when(k == 0)
    def _init():
        _store_x2h(r_ref[...], x2h_ref)

    _argmin_block(k, r_ref[...], w_ref, x2h_ref, y2h_ref, macc_ref, iacc_ref,
                  idx_ref)


def fused_body(rprev_ref, qprev_ref, zqprev_ref, w_ref,
               idx_ref, rnew_ref, zqnew_ref, lsum_ref,
               x2h_ref, y2h_ref, macc_ref, iacc_ref, lacc_ref):
    t = pl.program_id(0)
    k = pl.program_id(1)

    @pl.when(k == 0)
    def _update():
        rp = rprev_ref[...]
        q = qprev_ref[...]
        diff = rp - q
        part = jnp.sum(diff * diff)

        @pl.when(t == 0)
        def _l0():
            lacc_ref[0] = part

        @pl.when(t > 0)
        def _ln():
            lacc_ref[0] = lacc_ref[0] + part

        qs = rp + (q - rp)
        zqnew_ref[...] = zqprev_ref[...] + qs
        r = rp - qs
        rnew_ref[...] = r
        _store_x2h(r, x2h_ref)

    @pl.when((t == T // TM - 1) & (k == NKB - 1))
    def _lout():
        lsum_ref[0, 0] = lacc_ref[0]

    _argmin_block(k, rnew_ref[...], w_ref, x2h_ref, y2h_ref, macc_ref,
                  iacc_ref, idx_ref)


_scratch = [
    pltpu.VMEM((1, TM), jnp.float32),
    pltpu.VMEM((K, 1), jnp.float32),
    pltpu.VMEM((8, TM), jnp.float32),
    pltpu.VMEM((8, TM), jnp.float32),
]

_params = pltpu.CompilerParams(dimension_semantics=("arbitrary", "arbitrary"))

_IDX_SHAPE = jax.ShapeDtypeStruct((T // TM, 1, TM), jnp.int32)
_idx_spec = pl.BlockSpec((1, 1, TM), lambda t, k: (t, 0, 0))

layer0_call = pl.pallas_call(
    layer0_body,
    grid=(T // TM, NKB),
    in_specs=[
        pl.BlockSpec((TM, D), lambda t, k: (t, 0)),
        pl.BlockSpec((KB, D), lambda t, k: (k, 0)),
    ],
    out_specs=_idx_spec,
    out_shape=_IDX_SHAPE,
    scratch_shapes=_scratch,
    compiler_params=_params,
)

fused_call = pl.pallas_call(
    fused_body,
    grid=(T // TM, NKB),
    in_specs=[
        pl.BlockSpec((TM, D), lambda t, k: (t, 0)),
        pl.BlockSpec((TM, D), lambda t, k: (t, 0)),
        pl.BlockSpec((TM, D), lambda t, k: (t, 0)),
        pl.BlockSpec((KB, D), lambda t, k: (k, 0)),
    ],
    out_specs=[
        _idx_spec,
        pl.BlockSpec((TM, D), lambda t, k: (t, 0)),
        pl.BlockSpec((TM, D), lambda t, k: (t, 0)),
        pl.BlockSpec(memory_space=pltpu.SMEM),
    ],
    out_shape=[
        _IDX_SHAPE,
        jax.ShapeDtypeStruct((T, D), jnp.float32),
        jax.ShapeDtypeStruct((T, D), jnp.float32),
        jax.ShapeDtypeStruct((1, 1), jnp.float32),
    ],
    scratch_shapes=_scratch + [pltpu.SMEM((1,), jnp.float32)],
    compiler_params=_params,
)


@functools.lru_cache(maxsize=None)
def _sc_gather(off):
    @functools.partial(
        pl.kernel,
        mesh=plsc.VectorSubcoreMesh(core_axis_name="c", subcore_axis_name="s"),
        out_type=jax.ShapeDtypeStruct((T, D), jnp.float32),
        scratch_types=[
            pltpu.VMEM((BPW,), jnp.int32),
            pltpu.VMEM((BPW, D), jnp.float32),
            pltpu.SemaphoreType.DMA,
        ],
    )
    def sc_gather(cb_hbm, gidx_hbm, out_hbm, idx_v, rows_v, sem):
        wid = lax.axis_index("s") * 2 + lax.axis_index("c")
        base = wid * BPW
        pltpu.sync_copy(gidx_hbm.at[pl.ds(base, BPW)], idx_v)
        if off:
            for j in range(BPW // 16):
                sl = pl.ds(j * 16, 16)
                idx_v[sl] = idx_v[sl] + off
        pltpu.async_copy(cb_hbm.at[idx_v], rows_v, sem).wait()
        pltpu.sync_copy(rows_v, out_hbm.at[pl.ds(base, BPW)])

    return sc_gather


def kernel(z, codebooks):
    batch, dim, time = z.shape
    zt = jnp.transpose(z, (0, 2, 1))
    r0 = zt.reshape(T, D)
    cb_flat = codebooks.reshape(NL * K, D)
    idx = layer0_call(r0, codebooks[0]).reshape(T)
    q = _sc_gather(0)(cb_flat, idx)
    codes = [idx]
    r, zq = r0, jnp.zeros_like(r0)
    loss = jnp.zeros((), dtype=jnp.float32)
    inv_n = jnp.float32(1.0 / (T * D))
    for layer in range(1, NL):
        idx, r, zq, lsum = fused_call(r, q, zq, codebooks[layer])
        loss = loss + lsum[0, 0] * inv_n
        idx = idx.reshape(T)
        q = _sc_gather(layer * K)(cb_flat, idx)
        codes.append(idx)

    loss = loss + jnp.mean((r - q) ** 2)
    qs = r + (q - r)
    zq = zq + qs

    z_q_out = jnp.transpose(zq.reshape(batch, time, dim), (0, 2, 1))
    all_codes = jnp.stack([c.reshape(batch, time) for c in codes], axis=0)
    return (z_q_out, all_codes, loss, loss, loss + loss)
